# baseline (speedup 1.0000x reference)
import functools

import jax
import jax.numpy as jnp
from jax import lax
from jax.experimental import pallas as pl
from jax.experimental.pallas import tpu as pltpu
from jax.experimental.pallas import tpu_sc as plsc

N = 8192
D = 2048
DFF = 8192
KEEP = 4096

_RB = 512
_NRS = N // _RB
_NIS = KEEP // _RB
_W2CH = DFF // _NIS


def _rank_body(acol_ref, arow_ref, ranks_ref):
    def key(v):
        b = lax.bitcast_convert_type(v, jnp.int32)
        return jnp.where(b < 0, b ^ jnp.int32(0x7FFFFFFF), b)

    ai = key(acol_ref[...])
    aj = key(arow_ref[...])
    i0 = pl.program_id(0) * _RB
    ii = i0 + lax.broadcasted_iota(jnp.int32, (_RB, N), 0)
    jj = lax.broadcasted_iota(jnp.int32, (_RB, N), 1)
    before = (aj > ai) | ((aj == ai) & (jj < ii))
    ranks_ref[...] = jnp.sum(before.astype(jnp.int32), axis=1).reshape(1, 1, _RB)


def _compute_ranks(a):
    acol = a.reshape(N, 1)
    arow = a.reshape(1, N)
    ranks = pl.pallas_call(
        _rank_body,
        grid=(_NRS,),
        in_specs=[
            pl.BlockSpec((_RB, 1), lambda i: (i, 0)),
            pl.BlockSpec((1, N), lambda i: (0, 0)),
        ],
        out_specs=pl.BlockSpec((1, 1, _RB), lambda i: (i, 0, 0)),
        out_shape=jax.ShapeDtypeStruct((_NRS, 1, _RB), jnp.int32),
    )(acol, arow)
    return ranks.reshape(1, N)


def _invert_body(ranks_ref, w2_ref, idx_ref, w2b_ref):
    rr = ranks_ref[...]
    p0 = pl.program_id(0) * _RB
    pp = p0 + lax.broadcasted_iota(jnp.int32, (_RB, N), 0)
    ii = lax.broadcasted_iota(jnp.int32, (_RB, N), 1)
    hit = jnp.where(rr == pp, ii, 0)
    idx_ref[...] = jnp.sum(hit, axis=1).reshape(1, 1, _RB)
    w2b_ref[...] = w2_ref[...].astype(jnp.bfloat16)


def _invert_ranks(ranks, W2):
    idx, W2b = pl.pallas_call(
        _invert_body,
        grid=(_NIS,),
        in_specs=[
            pl.BlockSpec((1, N), lambda p: (0, 0)),
            pl.BlockSpec((_W2CH, D), lambda p: (p, 0)),
        ],
        out_specs=[
            pl.BlockSpec((1, 1, _RB), lambda p: (p, 0, 0)),
            pl.BlockSpec((_W2CH, D), lambda p: (p, 0)),
        ],
        out_shape=[
            jax.ShapeDtypeStruct((_NIS, 1, _RB), jnp.int32),
            jax.ShapeDtypeStruct((DFF, D), jnp.bfloat16),
        ],
    )(ranks, W2)
    return idx.reshape(KEEP), W2b


_NC = 2
_NS = 16
_NW = _NC * _NS
_RPW = KEEP // _NW
_CH = 16
_NCH = _RPW // _CH


def _sc_gather_body(idx_hbm, x_hbm, out_hbm, idx_v, buf0, buf1, sem0, sem1):
    wid = lax.axis_index("s") * _NC + lax.axis_index("c")
    lo = wid * _RPW

    pltpu.sync_copy(idx_hbm.at[pl.ds(lo, _RPW)], idx_v)

    bufs = (buf0, buf1)
    sems = (sem0, sem1)
    copies = [None, None]
    copies[0] = pltpu.async_copy(x_hbm.at[idx_v.at[pl.ds(0, _CH)]], bufs[0], sems[0])
    for c in range(_NCH):
        if c + 1 < _NCH:
            copies[(c + 1) % 2] = pltpu.async_copy(
                x_hbm.at[idx_v.at[pl.ds((c + 1) * _CH, _CH)]],
                bufs[(c + 1) % 2],
                sems[(c + 1) % 2],
            )
        copies[c % 2].wait()
        pltpu.sync_copy(bufs[c % 2], out_hbm.at[pl.ds(lo + c * _CH, _CH)])


@functools.cache
def _sc_gather_kernel():
    mesh = plsc.VectorSubcoreMesh(
        core_axis_name="c", subcore_axis_name="s", num_cores=_NC, num_subcores=_NS
    )
    return pl.kernel(
        _sc_gather_body,
        out_type=jax.ShapeDtypeStruct((KEEP, D), jnp.float32),
        mesh=mesh,
        scratch_types=[
            pltpu.VMEM((_RPW,), jnp.int32),
            pltpu.VMEM((_CH, D), jnp.float32),
            pltpu.VMEM((_CH, D), jnp.float32),
            pltpu.SemaphoreType.DMA,
            pltpu.SemaphoreType.DMA,
        ],
    )


def _sc_gather(top_idx, x):
    return _sc_gather_kernel()(top_idx, x)


_BN1 = 256
_BM2 = 256


def _mm1_body(xg_ref, w1_ref, h_ref):
    h = jnp.dot(xg_ref[...], w1_ref[...], preferred_element_type=jnp.float32)
    h_ref[...] = jnp.maximum(h, 0.0).astype(jnp.bfloat16)


def _mm1(xg, W1):
    return pl.pallas_call(
        _mm1_body,
        grid=(DFF // _BN1,),
        in_specs=[
            pl.BlockSpec((KEEP, D), lambda n: (0, 0)),
            pl.BlockSpec((D, _BN1), lambda n: (0, n)),
        ],
        out_specs=pl.BlockSpec((KEEP, _BN1), lambda n: (0, n)),
        out_shape=jax.ShapeDtypeStruct((KEEP, DFF), jnp.bfloat16),
    )(xg, W1)


def _mm2_body(h_ref, w2b_ref, o_ref):
    o_ref[...] = jnp.dot(h_ref[...], w2b_ref[...], preferred_element_type=jnp.float32)


def _mm2(h, W2b):
    return pl.pallas_call(
        _mm2_body,
        grid=(KEEP // _BM2,),
        in_specs=[
            pl.BlockSpec((_BM2, DFF), lambda m: (m, 0)),
            pl.BlockSpec((DFF, D), lambda m: (0, 0)),
        ],
        out_specs=pl.BlockSpec((_BM2, D), lambda m: (m, 0)),
        out_shape=jax.ShapeDtypeStruct((KEEP, D), jnp.float32),
    )(h, W2b)


def kernel(x, activations, W1, W2):
    ranks = _compute_ranks(activations)
    top_idx, W2b = _invert_ranks(ranks, W2)
    xg = _sc_gather(top_idx, x)
    h = _mm1(xg, W1)
    return _mm2(h, W2b)


# speedup vs baseline: 1.0051x; 1.0051x over previous
import functools

import jax
import jax.numpy as jnp
from jax import lax
from jax.experimental import pallas as pl
from jax.experimental.pallas import tpu as pltpu
from jax.experimental.pallas import tpu_sc as plsc

N = 8192
D = 2048
DFF = 8192
KEEP = 4096

_RB = 512
_NRS = N // _RB
_NIS = KEEP // _RB
_W2CH = DFF // _NIS


def _rankinv_body(acol_ref, arow_ref, w2_ref, idx_ref, w2b_ref, ranks_ref):
    step = pl.program_id(0)

    @pl.when(step < _NRS)
    def _rank_phase():
        def key(v):
            b = lax.bitcast_convert_type(v, jnp.int32)
            return jnp.where(b < 0, b ^ jnp.int32(0x7FFFFFFF), b)

        ai = key(acol_ref[...])
        aj = key(arow_ref[...])
        i0 = step * _RB
        ii = i0 + lax.broadcasted_iota(jnp.int32, (_RB, N), 0)
        jj = lax.broadcasted_iota(jnp.int32, (_RB, N), 1)
        before = (aj > ai) | ((aj == ai) & (jj < ii))
        ranks_ref[0, pl.ds(i0, _RB)] = jnp.sum(before.astype(jnp.int32), axis=1)

    @pl.when(step >= _NRS)
    def _invert_phase():
        rr = ranks_ref[...]
        p0 = (step - _NRS) * _RB
        pp = p0 + lax.broadcasted_iota(jnp.int32, (_RB, N), 0)
        ii = lax.broadcasted_iota(jnp.int32, (_RB, N), 1)
        hit = jnp.where(rr == pp, ii, 0)
        idx_ref[...] = jnp.sum(hit, axis=1).reshape(1, 1, _RB)
        w2b_ref[...] = w2_ref[...].astype(jnp.bfloat16)


def _rankinv(a, W2):
    acol = a.reshape(N, 1)
    arow = a.reshape(1, N)
    idx, W2b = pl.pallas_call(
        _rankinv_body,
        grid=(_NRS + _NIS,),
        in_specs=[
            pl.BlockSpec((_RB, 1), lambda i: (jnp.minimum(i, _NRS - 1), 0)),
            pl.BlockSpec((1, N), lambda i: (0, 0)),
            pl.BlockSpec((_W2CH, D), lambda i: (jnp.maximum(i - _NRS, 0), 0)),
        ],
        out_specs=[
            pl.BlockSpec((1, 1, _RB), lambda i: (jnp.maximum(i - _NRS, 0), 0, 0)),
            pl.BlockSpec((_W2CH, D), lambda i: (jnp.maximum(i - _NRS, 0), 0)),
        ],
        out_shape=[
            jax.ShapeDtypeStruct((_NIS, 1, _RB), jnp.int32),
            jax.ShapeDtypeStruct((DFF, D), jnp.bfloat16),
        ],
        scratch_shapes=[pltpu.VMEM((1, N), jnp.int32)],
    )(acol, arow, W2)
    return idx.reshape(KEEP), W2b


_NC = 2
_NS = 16
_NW = _NC * _NS
_RPW = KEEP // _NW
_CH = 16
_NCH = _RPW // _CH


def _sc_gather_body(idx_hbm, x_hbm, out_hbm, idx_v, buf0, buf1, sem0, sem1):
    wid = lax.axis_index("s") * _NC + lax.axis_index("c")
    lo = wid * _RPW

    pltpu.sync_copy(idx_hbm.at[pl.ds(lo, _RPW)], idx_v)

    bufs = (buf0, buf1)
    sems = (sem0, sem1)
    copies = [None, None]
    copies[0] = pltpu.async_copy(x_hbm.at[idx_v.at[pl.ds(0, _CH)]], bufs[0], sems[0])
    for c in range(_NCH):
        if c + 1 < _NCH:
            copies[(c + 1) % 2] = pltpu.async_copy(
                x_hbm.at[idx_v.at[pl.ds((c + 1) * _CH, _CH)]],
                bufs[(c + 1) % 2],
                sems[(c + 1) % 2],
            )
        copies[c % 2].wait()
        pltpu.sync_copy(bufs[c % 2], out_hbm.at[pl.ds(lo + c * _CH, _CH)])


@functools.cache
def _sc_gather_kernel():
    mesh = plsc.VectorSubcoreMesh(
        core_axis_name="c", subcore_axis_name="s", num_cores=_NC, num_subcores=_NS
    )
    return pl.kernel(
        _sc_gather_body,
        out_type=jax.ShapeDtypeStruct((KEEP, D), jnp.float32),
        mesh=mesh,
        scratch_types=[
            pltpu.VMEM((_RPW,), jnp.int32),
            pltpu.VMEM((_CH, D), jnp.float32),
            pltpu.VMEM((_CH, D), jnp.float32),
            pltpu.SemaphoreType.DMA,
            pltpu.SemaphoreType.DMA,
        ],
    )


def _sc_gather(top_idx, x):
    return _sc_gather_kernel()(top_idx, x)


_BN1 = 256
_BM2 = 256


def _mm1_body(xg_ref, w1_ref, h_ref):
    h = jnp.dot(xg_ref[...], w1_ref[...], preferred_element_type=jnp.float32)
    h_ref[...] = jnp.maximum(h, 0.0).astype(jnp.bfloat16)


def _mm1(xg, W1):
    return pl.pallas_call(
        _mm1_body,
        grid=(DFF // _BN1,),
        in_specs=[
            pl.BlockSpec((KEEP, D), lambda n: (0, 0)),
            pl.BlockSpec((D, _BN1), lambda n: (0, n)),
        ],
        out_specs=pl.BlockSpec((KEEP, _BN1), lambda n: (0, n)),
        out_shape=jax.ShapeDtypeStruct((KEEP, DFF), jnp.bfloat16),
    )(xg, W1)


def _mm2_body(h_ref, w2b_ref, o_ref):
    o_ref[...] = jnp.dot(h_ref[...], w2b_ref[...], preferred_element_type=jnp.float32)


def _mm2(h, W2b):
    return pl.pallas_call(
        _mm2_body,
        grid=(KEEP // _BM2,),
        in_specs=[
            pl.BlockSpec((_BM2, DFF), lambda m: (m, 0)),
            pl.BlockSpec((DFF, D), lambda m: (0, 0)),
        ],
        out_specs=pl.BlockSpec((_BM2, D), lambda m: (m, 0)),
        out_shape=jax.ShapeDtypeStruct((KEEP, D), jnp.float32),
    )(h, W2b)


def kernel(x, activations, W1, W2):
    top_idx, W2b = _rankinv(activations, W2)
    xg = _sc_gather(top_idx, x)
    h = _mm1(xg, W1)
    return _mm2(h, W2b)


# speedup vs baseline: 1.0096x; 1.0044x over previous
import functools

import jax
import jax.numpy as jnp
from jax import lax
from jax.experimental import pallas as pl
from jax.experimental.pallas import tpu as pltpu
from jax.experimental.pallas import tpu_sc as plsc

N = 8192
D = 2048
DFF = 8192
KEEP = 4096

_RB = 512
_NRS = N // _RB
_NIS = KEEP // _RB
_W2CH = DFF // _NIS


def _rankinv_body(acol_ref, arow_ref, w2_ref, idx_ref, w2b_ref, ranks_ref):
    step = pl.program_id(0)

    @pl.when(step < _NRS)
    def _rank_phase():
        def key(v):
            b = lax.bitcast_convert_type(v, jnp.int32)
            return jnp.where(b < 0, b ^ jnp.int32(0x7FFFFFFF), b)

        ai = key(acol_ref[...])
        aj = key(arow_ref[...])
        i0 = step * _RB
        ii = i0 + lax.broadcasted_iota(jnp.int32, (_RB, N), 0)
        jj = lax.broadcasted_iota(jnp.int32, (_RB, N), 1)
        before = (aj > ai) | ((aj == ai) & (jj < ii))
        ranks_ref[0, pl.ds(i0, _RB)] = jnp.sum(before.astype(jnp.int32), axis=1)

    @pl.when(step >= _NRS)
    def _invert_phase():
        rr = ranks_ref[...]
        p0 = (step - _NRS) * _RB
        pp = p0 + lax.broadcasted_iota(jnp.int32, (_RB, N), 0)
        ii = lax.broadcasted_iota(jnp.int32, (_RB, N), 1)
        hit = jnp.where(rr == pp, ii, 0)
        idx_ref[...] = jnp.sum(hit, axis=1).reshape(1, 1, _RB)
        w2b_ref[...] = w2_ref[...].astype(jnp.bfloat16)


def _rankinv(a, W2):
    acol = a.reshape(N, 1)
    arow = a.reshape(1, N)
    idx, W2b = pl.pallas_call(
        _rankinv_body,
        grid=(_NRS + _NIS,),
        in_specs=[
            pl.BlockSpec((_RB, 1), lambda i: (jnp.minimum(i, _NRS - 1), 0)),
            pl.BlockSpec((1, N), lambda i: (0, 0)),
            pl.BlockSpec((_W2CH, D), lambda i: (jnp.maximum(i - _NRS, 0), 0)),
        ],
        out_specs=[
            pl.BlockSpec((1, 1, _RB), lambda i: (jnp.maximum(i - _NRS, 0), 0, 0)),
            pl.BlockSpec((_W2CH, D), lambda i: (jnp.maximum(i - _NRS, 0), 0)),
        ],
        out_shape=[
            jax.ShapeDtypeStruct((_NIS, 1, _RB), jnp.int32),
            jax.ShapeDtypeStruct((DFF, D), jnp.bfloat16),
        ],
        scratch_shapes=[pltpu.VMEM((1, N), jnp.int32)],
    )(acol, arow, W2)
    return idx.reshape(KEEP), W2b


_NC = 2
_NS = 16
_NW = _NC * _NS
_RPW = KEEP // _NW
_CH = 16
_NCH = _RPW // _CH


def _sc_gather_body(idx_hbm, x_hbm, out_hbm, idx_v, buf0, buf1, sem0, sem1):
    wid = lax.axis_index("s") * _NC + lax.axis_index("c")
    lo = wid * _RPW

    pltpu.sync_copy(idx_hbm.at[pl.ds(lo, _RPW)], idx_v)

    bufs = (buf0, buf1)
    sems = (sem0, sem1)
    copies = [None, None]
    copies[0] = pltpu.async_copy(x_hbm.at[idx_v.at[pl.ds(0, _CH)]], bufs[0], sems[0])
    for c in range(_NCH):
        if c + 1 < _NCH:
            copies[(c + 1) % 2] = pltpu.async_copy(
                x_hbm.at[idx_v.at[pl.ds((c + 1) * _CH, _CH)]],
                bufs[(c + 1) % 2],
                sems[(c + 1) % 2],
            )
        copies[c % 2].wait()
        pltpu.sync_copy(bufs[c % 2], out_hbm.at[pl.ds(lo + c * _CH, _CH)])


@functools.cache
def _sc_gather_kernel():
    mesh = plsc.VectorSubcoreMesh(
        core_axis_name="c", subcore_axis_name="s", num_cores=_NC, num_subcores=_NS
    )
    return pl.kernel(
        _sc_gather_body,
        out_type=jax.ShapeDtypeStruct((KEEP, D), jnp.float32),
        mesh=mesh,
        scratch_types=[
            pltpu.VMEM((_RPW,), jnp.int32),
            pltpu.VMEM((_CH, D), jnp.float32),
            pltpu.VMEM((_CH, D), jnp.float32),
            pltpu.SemaphoreType.DMA,
            pltpu.SemaphoreType.DMA,
        ],
    )


def _sc_gather(top_idx, x):
    return _sc_gather_kernel()(top_idx, x)


_BN1 = 512
_BM2 = 256


def _mm1_body(xg_ref, w1_ref, h_ref):
    h = jnp.dot(xg_ref[...], w1_ref[...], preferred_element_type=jnp.float32)
    h_ref[...] = jnp.maximum(h, 0.0).astype(jnp.bfloat16)


def _mm1(xg, W1):
    return pl.pallas_call(
        _mm1_body,
        grid=(DFF // _BN1,),
        in_specs=[
            pl.BlockSpec((KEEP, D), lambda n: (0, 0)),
            pl.BlockSpec((D, _BN1), lambda n: (0, n)),
        ],
        out_specs=pl.BlockSpec((KEEP, _BN1), lambda n: (0, n)),
        out_shape=jax.ShapeDtypeStruct((KEEP, DFF), jnp.bfloat16),
    )(xg, W1)


def _mm2_body(h_ref, w2b_ref, o_ref):
    o_ref[...] = jnp.dot(h_ref[...], w2b_ref[...], preferred_element_type=jnp.float32)


def _mm2(h, W2b):
    return pl.pallas_call(
        _mm2_body,
        grid=(KEEP // _BM2,),
        in_specs=[
            pl.BlockSpec((_BM2, DFF), lambda m: (m, 0)),
            pl.BlockSpec((DFF, D), lambda m: (0, 0)),
        ],
        out_specs=pl.BlockSpec((_BM2, D), lambda m: (m, 0)),
        out_shape=jax.ShapeDtypeStruct((KEEP, D), jnp.float32),
    )(h, W2b)


def kernel(x, activations, W1, W2):
    top_idx, W2b = _rankinv(activations, W2)
    xg = _sc_gather(top_idx, x)
    h = _mm1(xg, W1)
    return _mm2(h, W2b)
